# Initial kernel scaffold; baseline (speedup 1.0000x reference)
import jax
import jax.numpy as jnp
from jax.experimental import pallas as pl


def kernel(x, edge_index, edges_type, W_rel, W_self, b_rel, W1, b1, W2, b2):
    raise NotImplementedError("write your pallas kernel here")


# speedup vs baseline: 28.5584x; 28.5584x over previous
import functools

import jax
import jax.numpy as jnp
from jax import lax
from jax.experimental import pallas as pl
from jax.experimental.pallas import tpu as pltpu
from jax.experimental.pallas import tpu_sc as plsc

N = 10000
E = 320000
D = 128
H = 128
O = 128
R = 8

NC = 2
NS = 16
NW = NC * NS
EPW = E // NW
CH = 80
KCH = EPW // CH
NPAD = 10240
RPT = NPAD // NS


def _table_block(x_ref, w_ref, src_ref, et_ref, o_ref, comb_ref):
    o_ref[...] = jnp.dot(x_ref[...], w_ref[...],
                         preferred_element_type=jnp.float32)
    comb_ref[...] = src_ref[...] * R + et_ref[...]


def _compute_tables(x, w_rel_flat, src2, et2, bn=1000):
    er = E // 128
    return pl.pallas_call(
        _table_block,
        grid=(N // bn,),
        in_specs=[
            pl.BlockSpec((bn, D), lambda i: (i, 0)),
            pl.BlockSpec((D, R * O), lambda i: (0, 0)),
            pl.BlockSpec((er, 128), lambda i: (0, 0)),
            pl.BlockSpec((er, 128), lambda i: (0, 0)),
        ],
        out_specs=[
            pl.BlockSpec((bn, R * O), lambda i: (i, 0)),
            pl.BlockSpec((er, 128), lambda i: (0, 0)),
        ],
        out_shape=[
            jax.ShapeDtypeStruct((N, R * O), jnp.float32),
            jax.ShapeDtypeStruct((er, 128), jnp.int32),
        ],
    )(x, w_rel_flat, src2, et2)


_MESH = plsc.VectorSubcoreMesh(core_axis_name="c", subcore_axis_name="s",
                               num_cores=NC, num_subcores=NS)


@functools.partial(
    pl.kernel,
    out_type=jax.ShapeDtypeStruct((NC, NPAD, O), jnp.float32),
    mesh=_MESH,
    scratch_types=[
        pltpu.VMEM((EPW,), jnp.int32),
        pltpu.VMEM((KCH, CH), jnp.int32),
        pltpu.VMEM((2, CH, O), jnp.float32),
        pltpu.VMEM_SHARED((NPAD, O), jnp.float32),
        pltpu.SemaphoreType.DMA((2,)),
        pltpu.SemaphoreType.DMA,
    ],
)
def _edge_agg(table_hbm, comb_hbm, dst_hbm, zeros_hbm, out_hbm,
              comb_s, dst_s, rows, acc_sh, gsem, ssem):
    cid = lax.axis_index("c")
    sid = lax.axis_index("s")
    wid = sid * NC + cid
    r0 = sid * RPT
    c_comb = pltpu.async_copy(comb_hbm.at[wid], comb_s, ssem)
    c_dst = pltpu.async_copy(dst_hbm.at[wid], dst_s, ssem)
    pltpu.sync_copy(zeros_hbm.at[pl.ds(r0, RPT)], acc_sh.at[pl.ds(r0, RPT)])
    c_comb.wait()
    c_dst.wait()
    plsc.subcore_barrier()

    def _gather(k, b):
        return pltpu.make_async_copy(
            table_hbm.at[comb_s.at[pl.ds(k * CH, CH)]], rows.at[b],
            gsem.at[b])

    _gather(0, 0).start()
    _gather(1, 1).start()

    @pl.loop(0, KCH - 1, step=2)
    def _grp(k0):
        for b in range(2):
            k = k0 + b
            _gather(k, b).wait()
            pltpu.sync_copy(rows.at[b], acc_sh.at[dst_s.at[k]], add=True)

            @pl.when(k + 2 < KCH)
            def _():
                _gather(k + 2, b).start()

    _gather(KCH - 1, 0).wait()
    pltpu.sync_copy(rows.at[0], acc_sh.at[dst_s.at[KCH - 1]], add=True)

    plsc.subcore_barrier()
    pltpu.sync_copy(acc_sh.at[pl.ds(r0, RPT)],
                    out_hbm.at[cid, pl.ds(r0, RPT)])


def _mlp_block(x_ref, agg_ref, wself_ref, brel_ref, w1x_ref, w1m_ref, b1_ref,
               w2x_ref, w2m_ref, b2_ref, o_ref):
    x = x_ref[...]
    msg = (agg_ref[0] + agg_ref[1] + brel_ref[...]
           + jnp.dot(x, wself_ref[...], preferred_element_type=jnp.float32))
    mid = jnp.tanh(
        jnp.dot(x, w1x_ref[...], preferred_element_type=jnp.float32)
        + jnp.dot(msg, w1m_ref[...], preferred_element_type=jnp.float32)
        + b1_ref[...])
    o_ref[...] = (
        jnp.dot(x, w2x_ref[...], preferred_element_type=jnp.float32)
        + jnp.dot(mid, w2m_ref[...], preferred_element_type=jnp.float32)
        + b2_ref[...])


def _mlp(x, agg, W_self, b_rel, W1, b1, W2, b2, bn=1000):
    w1x, w1m = W1[:D], W1[D:]
    w2x, w2m = W2[:D], W2[D:]
    full = lambda r, c: pl.BlockSpec((r, c), lambda i: (0, 0))
    return pl.pallas_call(
        _mlp_block,
        grid=(N // bn,),
        in_specs=[
            pl.BlockSpec((bn, D), lambda i: (i, 0)),
            pl.BlockSpec((NC, bn, O), lambda i: (0, i, 0)),
            full(D, O), full(1, O), full(D, 2 * H), full(H, 2 * H),
            full(1, 2 * H), full(D, O), full(2 * H, O), full(1, O),
        ],
        out_specs=pl.BlockSpec((bn, O), lambda i: (i, 0)),
        out_shape=jax.ShapeDtypeStruct((N, O), jnp.float32),
    )(x, agg, W_self, b_rel.reshape(1, O), w1x, w1m, b1.reshape(1, 2 * H),
      w2x, w2m, b2.reshape(1, O))


def kernel(x, edge_index, edges_type, W_rel, W_self, b_rel, W1, b1, W2, b2):
    w_rel_flat = jnp.transpose(W_rel, (1, 0, 2)).reshape(D, R * O)
    src2 = edge_index[0].reshape(E // 128, 128)
    et2 = edges_type.reshape(E // 128, 128)

    tables, comb2 = _compute_tables(x, w_rel_flat, src2, et2)
    xw_table = tables.reshape(N * R, O)

    zeros = jnp.zeros((NPAD, O), jnp.float32)
    comb_w = comb2.reshape(NW, EPW)
    dst3 = edge_index[1].reshape(NW, KCH, CH)
    agg = _edge_agg(xw_table, comb_w, dst3, zeros)

    return _mlp(x, agg, W_self, b_rel, W1, b1, W2, b2)


# speedup vs baseline: 31.6834x; 1.1094x over previous
import functools

import jax
import jax.numpy as jnp
from jax import lax
from jax.experimental import pallas as pl
from jax.experimental.pallas import tpu as pltpu
from jax.experimental.pallas import tpu_sc as plsc

N = 10000
E = 320000
D = 128
H = 128
O = 128
R = 8

NC = 2
NS = 16
NW = NC * NS
EPW = E // NW
CH = 80
KCH = EPW // CH
NPAD = 10240
RPT = NPAD // NS


def _table_block(x_ref, w_ref, o_ref):
    o_ref[...] = jnp.dot(x_ref[...], w_ref[0],
                         preferred_element_type=jnp.float32)


def _compute_table(x, W_rel, bn=2000):
    nb = N // bn
    return pl.pallas_call(
        _table_block,
        grid=(nb, R),
        in_specs=[
            pl.BlockSpec((bn, D), lambda i, r: (i, 0)),
            pl.BlockSpec((1, D, O), lambda i, r: (r, 0, 0)),
        ],
        out_specs=pl.BlockSpec((bn, O), lambda i, r: (r * nb + i, 0)),
        out_shape=jax.ShapeDtypeStruct((R * N, O), jnp.float32),
    )(x, W_rel)


def _comb_block(src_ref, et_ref, comb_ref):
    comb_ref[...] = et_ref[...] * N + src_ref[...]


def _compute_comb(src2, et2):
    er = E // 128
    full = pl.BlockSpec((er, 128), lambda: (0, 0))
    return pl.pallas_call(
        _comb_block,
        in_specs=[full, full],
        out_specs=full,
        out_shape=jax.ShapeDtypeStruct((er, 128), jnp.int32),
    )(src2, et2)


_MESH = plsc.VectorSubcoreMesh(core_axis_name="c", subcore_axis_name="s",
                               num_cores=NC, num_subcores=NS)


@functools.partial(
    pl.kernel,
    out_type=jax.ShapeDtypeStruct((NC, NPAD, O), jnp.float32),
    mesh=_MESH,
    scratch_types=[
        pltpu.VMEM((EPW,), jnp.int32),
        pltpu.VMEM((KCH, CH), jnp.int32),
        pltpu.VMEM((2, CH, O), jnp.float32),
        pltpu.VMEM_SHARED((NPAD, O), jnp.float32),
        pltpu.SemaphoreType.DMA((2,)),
        pltpu.SemaphoreType.DMA,
    ],
)
def _edge_agg(table_hbm, comb_hbm, dst_hbm, zeros_hbm, out_hbm,
              comb_s, dst_s, rows, acc_sh, gsem, ssem):
    cid = lax.axis_index("c")
    sid = lax.axis_index("s")
    wid = sid * NC + cid
    r0 = sid * RPT
    c_comb = pltpu.async_copy(comb_hbm.at[wid], comb_s, ssem)
    c_dst = pltpu.async_copy(dst_hbm.at[wid], dst_s, ssem)
    pltpu.sync_copy(zeros_hbm.at[pl.ds(r0, RPT)], acc_sh.at[pl.ds(r0, RPT)])
    c_comb.wait()
    c_dst.wait()
    plsc.subcore_barrier()

    def _gather(k, b):
        return pltpu.make_async_copy(
            table_hbm.at[comb_s.at[pl.ds(k * CH, CH)]], rows.at[b],
            gsem.at[b])

    _gather(0, 0).start()
    _gather(1, 1).start()

    @pl.loop(0, KCH - 1, step=2)
    def _grp(k0):
        for b in range(2):
            k = k0 + b
            _gather(k, b).wait()
            pltpu.sync_copy(rows.at[b], acc_sh.at[dst_s.at[k]], add=True)

            @pl.when(k + 2 < KCH)
            def _():
                _gather(k + 2, b).start()

    _gather(KCH - 1, 0).wait()
    pltpu.sync_copy(rows.at[0], acc_sh.at[dst_s.at[KCH - 1]], add=True)

    plsc.subcore_barrier()
    pltpu.sync_copy(acc_sh.at[pl.ds(r0, RPT)],
                    out_hbm.at[cid, pl.ds(r0, RPT)])


def _mlp_block(x_ref, agg_ref, wself_ref, brel_ref, w1x_ref, w1m_ref, b1_ref,
               w2x_ref, w2m_ref, b2_ref, o_ref):
    x = x_ref[...]
    msg = (agg_ref[0] + agg_ref[1] + brel_ref[...]
           + jnp.dot(x, wself_ref[...], preferred_element_type=jnp.float32))
    mid = jnp.tanh(
        jnp.dot(x, w1x_ref[...], preferred_element_type=jnp.float32)
        + jnp.dot(msg, w1m_ref[...], preferred_element_type=jnp.float32)
        + b1_ref[...])
    o_ref[...] = (
        jnp.dot(x, w2x_ref[...], preferred_element_type=jnp.float32)
        + jnp.dot(mid, w2m_ref[...], preferred_element_type=jnp.float32)
        + b2_ref[...])


def _mlp(x, agg, W_self, b_rel, W1, b1, W2, b2, bn=1000):
    w1x, w1m = W1[:D], W1[D:]
    w2x, w2m = W2[:D], W2[D:]
    full = lambda r, c: pl.BlockSpec((r, c), lambda i: (0, 0))
    return pl.pallas_call(
        _mlp_block,
        grid=(N // bn,),
        in_specs=[
            pl.BlockSpec((bn, D), lambda i: (i, 0)),
            pl.BlockSpec((NC, bn, O), lambda i: (0, i, 0)),
            full(D, O), full(1, O), full(D, 2 * H), full(H, 2 * H),
            full(1, 2 * H), full(D, O), full(2 * H, O), full(1, O),
        ],
        out_specs=pl.BlockSpec((bn, O), lambda i: (i, 0)),
        out_shape=jax.ShapeDtypeStruct((N, O), jnp.float32),
    )(x, agg, W_self, b_rel.reshape(1, O), w1x, w1m, b1.reshape(1, 2 * H),
      w2x, w2m, b2.reshape(1, O))


def kernel(x, edge_index, edges_type, W_rel, W_self, b_rel, W1, b1, W2, b2):
    src2 = edge_index[0].reshape(E // 128, 128)
    et2 = edges_type.reshape(E // 128, 128)

    xw_table = _compute_table(x, W_rel)
    comb2 = _compute_comb(src2, et2)

    zeros = jnp.zeros((NPAD, O), jnp.float32)
    comb_w = comb2.reshape(NW, EPW)
    dst3 = edge_index[1].reshape(NW, KCH, CH)
    agg = _edge_agg(xw_table, comb_w, dst3, zeros)

    return _mlp(x, agg, W_self, b_rel, W1, b1, W2, b2)
